# baseline (speedup 1.0000x reference)
import functools

import jax
import jax.numpy as jnp
from jax import lax
from jax.experimental import pallas as pl
from jax.experimental.pallas import tpu as pltpu
from jax.experimental.pallas import tpu_sc as plsc

NUM_A = 16384
NUM_B = 26
NUM_ROWS = NUM_A * NUM_B
DIM = 32
NUM_S = 1000000
NC, NS = 2, 16
NW = NC * NS
ROWS_PER_W = NUM_ROWS // NW
CHUNK = 128
NCHUNK = ROWS_PER_W // CHUNK
NBUF = 2
MAIN = NCHUNK - NBUF

NTC = 7813
NTC_FULL = NTC - 1
BASE_CNT = NTC_FULL // NW
REM = NTC_FULL - BASE_CNT * NW


def _body_a(tabT_hbm, tail_hbm, lin_hbm, in_buf, out_c, tail_v, g0, g1, s0, s1):
    gsem = (g0, g1)
    ssem = (s0, s1)
    w = lax.axis_index("s") * NC + lax.axis_index("c")
    start = w * BASE_CNT + jnp.minimum(w, REM)
    count = BASE_CNT + jnp.where(w < REM, 1, 0)
    lane = lax.iota(jnp.int32, 16)

    def fire(tc, b):
        for jr in range(4):
            pltpu.async_copy(
                tabT_hbm.at[pl.ds(8 * jr, 8), pl.ds(128 * tc, 128)],
                in_buf.at[b].at[pl.ds(8 * jr, 8), pl.ds(0, 128)], gsem[b])

    def drain_in(tc, b):
        for jr in range(4):
            pltpu.make_async_copy(
                tabT_hbm.at[pl.ds(8 * jr, 8), pl.ds(128 * tc, 128)],
                in_buf.at[b].at[pl.ds(8 * jr, 8), pl.ds(0, 128)],
                gsem[b]).wait()

    def store(tc, b):
        pltpu.async_copy(out_c.at[b], lin_hbm.at[pl.ds(32 * tc, 32)], ssem[b])

    def drain_store(tc, b):
        pltpu.make_async_copy(
            out_c.at[b], lin_hbm.at[pl.ds(32 * tc, 32)], ssem[b]).wait()

    def transpose(b, nr):
        for r in range(nr):
            for q in range(8):
                row = lane + (16 * (q % 2))
                col = jnp.full((16,), 4 * r + q // 2, jnp.int32)
                v = plsc.load_gather(in_buf.at[b], [row, col])
                out_c.at[b][r, pl.ds(16 * q, 16)] = v

    def step(t):
        for b in range(NBUF):
            tc = start + t + b

            @pl.when(t + b < count)
            def _():
                drain_in(tc, b)

                @pl.when(t + b >= NBUF)
                def _():
                    drain_store(tc - NBUF, b)

                transpose(b, 32)

                @pl.when(t + b + NBUF < count)
                def _():
                    fire(tc + NBUF, b)

                store(tc, b)

    for b in range(NBUF):
        @pl.when(b < count)
        def _():
            fire(start + b, b)

    def _loop_body(i, carry):
        step(i * NBUF)
        return carry

    lax.fori_loop(0, (count + NBUF - 1) // NBUF, _loop_body, 0)

    for b in range(NBUF):
        lb = count - 1 - ((count - 1 - b) % NBUF)
        drain_store(start + lb, b)

    @pl.when(w == NW - 1)
    def _():
        pltpu.sync_copy(tail_hbm, tail_v)
        pltpu.sync_copy(tail_v, lin_hbm.at[pl.ds(32 * NTC_FULL, 16)])


def _body_b(idx_hbm, table_hbm, out5_hbm, idx_v, rows_v, trans_v, g0, g1, s0, s1):
    gsem = (g0, g1)
    ssem = (s0, s1)
    w = lax.axis_index("s") * NC + lax.axis_index("c")
    pltpu.sync_copy(idx_hbm.at[w], idx_v)
    cbase = w * NCHUNK
    lane = lax.iota(jnp.int32, 16)
    lane16 = lane + 16

    def fire(jc, p):
        pltpu.async_copy(table_hbm.at[idx_v.at[jc]], rows_v.at[p], gsem[p])

    def drain_gather(jc, p):
        pltpu.make_async_copy(
            table_hbm.at[idx_v.at[jc]], rows_v.at[p], gsem[p]).wait()

    def store(jc, p):
        c = cbase + jc
        b1 = c // 128
        ac = c % 128
        for jr in range(4):
            pltpu.async_copy(
                trans_v.at[p].at[pl.ds(8 * jr, 8), pl.ds(0, CHUNK)],
                out5_hbm.at[b1].at[jr].at[ac], ssem[p])

    def drain_store(jc, p):
        c = cbase + jc
        b1 = c // 128
        ac = c % 128
        for jr in range(4):
            pltpu.make_async_copy(
                trans_v.at[p].at[pl.ds(8 * jr, 8), pl.ds(0, CHUNK)],
                out5_hbm.at[b1].at[jr].at[ac], ssem[p]).wait()

    def transpose(p):
        def astep(a0):
            for u in range(8):
                aa = a0 + u
                col = jnp.full((16,), aa, jnp.int32)
                v0 = rows_v.at[p][aa, pl.ds(0, 16)]
                v1 = rows_v.at[p][aa, pl.ds(16, 16)]
                plsc.store_scatter(trans_v.at[p], [lane, col], v0)
                plsc.store_scatter(trans_v.at[p], [lane16, col], v1)

        pl.loop(0, CHUNK, step=8)(astep)

    def process(jc, p, fire_next, drain_prev):
        drain_gather(jc, p)
        if drain_prev:
            drain_store(jc - NBUF, p)
        transpose(p)
        if fire_next:
            fire(jc + NBUF, p)
        store(jc, p)

    for p in range(NBUF):
        fire(p, p)

    def grp(g):
        for p in range(NBUF):
            process(g + p, p, True, True)

    def grp_head(g):
        for p in range(NBUF):
            process(g + p, p, True, False)

    grp_head(0)
    pl.loop(NBUF, MAIN, step=NBUF)(grp)

    for p in range(NBUF):
        process(MAIN + p, p, False, True)
    for p in range(NBUF):
        drain_store(MAIN + p, p)


@jax.jit
def _run(idx3, tabT, tail3):
    ka = pl.kernel(
        _body_a,
        out_type=jax.ShapeDtypeStruct((NUM_S // 4, 128), jnp.float32),
        mesh=plsc.VectorSubcoreMesh(core_axis_name="c", subcore_axis_name="s"),
        scratch_types=[
            pltpu.VMEM((NBUF, DIM, 129), jnp.float32),
            pltpu.VMEM((NBUF, DIM, 128), jnp.float32),
            pltpu.VMEM((16, 128), jnp.float32),
            pltpu.SemaphoreType.DMA,
            pltpu.SemaphoreType.DMA,
            pltpu.SemaphoreType.DMA,
            pltpu.SemaphoreType.DMA,
        ],
        compiler_params=pltpu.CompilerParams(
            use_tc_tiling_on_sc=True, needs_layout_passes=False),
    )
    lin = ka(tabT, tail3)
    table = lin.reshape(NUM_S, DIM)
    kb = pl.kernel(
        _body_b,
        out_type=jax.ShapeDtypeStruct((NUM_B, 4, 128, 8, 128), jnp.float32),
        mesh=plsc.VectorSubcoreMesh(core_axis_name="c", subcore_axis_name="s"),
        scratch_types=[
            pltpu.VMEM((NCHUNK, CHUNK), jnp.int32),
            pltpu.VMEM((NBUF, CHUNK, DIM), jnp.float32),
            pltpu.VMEM((NBUF, DIM, 129), jnp.float32),
            pltpu.SemaphoreType.DMA,
            pltpu.SemaphoreType.DMA,
            pltpu.SemaphoreType.DMA,
            pltpu.SemaphoreType.DMA,
        ],
        compiler_params=pltpu.CompilerParams(
            use_tc_tiling_on_sc=False, needs_layout_passes=False),
    )
    return kb(idx3, table)


def kernel(inputs, table):
    idx3 = inputs.astype(jnp.int32).T.reshape(NW, NCHUNK, CHUNK)
    tail3 = table[NUM_S - 64:].reshape(16, 128)
    out5 = _run(idx3, table.T, tail3)
    return out5.transpose(2, 4, 0, 1, 3).reshape(NUM_A, NUM_B, DIM)


# speedup vs baseline: 1.8235x; 1.8235x over previous
import functools

import jax
import jax.numpy as jnp
from jax import lax
from jax.experimental import pallas as pl
from jax.experimental.pallas import tpu as pltpu
from jax.experimental.pallas import tpu_sc as plsc

NUM_A = 16384
NUM_B = 26
NUM_ROWS = NUM_A * NUM_B
DIM = 32
NC, NS = 2, 16
NW = NC * NS
ROWS_PER_W = NUM_ROWS // NW
CHUNK = 128
NCHUNK = ROWS_PER_W // CHUNK
NBUF = 2
MAIN = NCHUNK - NBUF


def _body(idx_hbm, table_hbm, out5_hbm, idx_v, rows_v, trans_v, g0, g1, s0, s1):
    gsem = (g0, g1)
    ssem = (s0, s1)
    w = lax.axis_index("s") * NC + lax.axis_index("c")
    pltpu.sync_copy(idx_hbm.at[w], idx_v)
    cbase = w * NCHUNK
    lane = lax.iota(jnp.int32, 16)

    def fire(jc, p):
        pltpu.async_copy(table_hbm.at[idx_v.at[jc]], rows_v.at[p], gsem[p])

    def drain_gather(jc, p):
        pltpu.make_async_copy(
            table_hbm.at[idx_v.at[jc]], rows_v.at[p], gsem[p]).wait()

    def store(jc, p):
        c = cbase + jc
        b1 = c // 128
        ac = c % 128
        for jr in range(4):
            pltpu.async_copy(
                trans_v.at[p].at[pl.ds(8 * jr, 8), pl.ds(0, CHUNK)],
                out5_hbm.at[b1].at[jr].at[ac], ssem[p])

    def drain_store(jc, p):
        c = cbase + jc
        b1 = c // 128
        ac = c % 128
        for jr in range(4):
            pltpu.make_async_copy(
                trans_v.at[p].at[pl.ds(8 * jr, 8), pl.ds(0, CHUNK)],
                out5_hbm.at[b1].at[jr].at[ac], ssem[p]).wait()

    lane16 = lane + 16

    def transpose(p):
        def astep(a0):
            for u in range(8):
                aa = a0 + u
                col = jnp.full((16,), aa, jnp.int32)
                v0 = rows_v.at[p][aa, pl.ds(0, 16)]
                v1 = rows_v.at[p][aa, pl.ds(16, 16)]
                plsc.store_scatter(trans_v.at[p], [lane, col], v0)
                plsc.store_scatter(trans_v.at[p], [lane16, col], v1)

        pl.loop(0, CHUNK, step=8)(astep)

    def process(jc, p, fire_next, drain_prev):
        drain_gather(jc, p)
        if drain_prev:
            drain_store(jc - NBUF, p)
        transpose(p)
        if fire_next:
            fire(jc + NBUF, p)
        store(jc, p)

    for p in range(NBUF):
        fire(p, p)

    def grp(g):
        for p in range(NBUF):
            process(g + p, p, True, True)

    def grp_head(g):
        for p in range(NBUF):
            process(g + p, p, True, False)

    grp_head(0)
    pl.loop(NBUF, MAIN, step=NBUF)(grp)

    for p in range(NBUF):
        process(MAIN + p, p, False, True)
    for p in range(NBUF):
        drain_store(MAIN + p, p)


@jax.jit
def _run(idx3, table):
    k = pl.kernel(
        _body,
        out_type=jax.ShapeDtypeStruct((NUM_B, 4, 128, 8, 128), jnp.float32),
        mesh=plsc.VectorSubcoreMesh(core_axis_name="c", subcore_axis_name="s"),
        scratch_types=[
            pltpu.VMEM((NCHUNK, CHUNK), jnp.int32),
            pltpu.VMEM((NBUF, CHUNK, DIM), jnp.float32),
            pltpu.VMEM((NBUF, DIM, 129), jnp.float32),
            pltpu.SemaphoreType.DMA,
            pltpu.SemaphoreType.DMA,
            pltpu.SemaphoreType.DMA,
            pltpu.SemaphoreType.DMA,
        ],
        compiler_params=pltpu.CompilerParams(use_tc_tiling_on_sc=False, needs_layout_passes=False),
    )
    return k(idx3, table)


def kernel(inputs, table):
    idx3 = inputs.astype(jnp.int32).T.reshape(NW, NCHUNK, CHUNK)
    out5 = _run(idx3, table)
    return out5.transpose(2, 4, 0, 1, 3).reshape(NUM_A, NUM_B, DIM)
